# speedup vs baseline: 3.2593x; 3.2593x over previous
import functools

import jax
import jax.numpy as jnp
from jax import lax
from jax.experimental import pallas as pl
from jax.experimental.pallas import tpu as pltpu
from jax.experimental.pallas import tpu_sc as plsc

V = 1024
D = 384
B = 131072
NC = 2
NS = 16
NW = NC * NS
BPW = B // NW
CHUNK = 128
NCH = BPW // CHUNK


def _sc_gather(position_ids, pos_embed):
    mesh = plsc.VectorSubcoreMesh(core_axis_name="c", subcore_axis_name="s")

    @functools.partial(
        pl.kernel,
        mesh=mesh,
        out_type=jax.ShapeDtypeStruct((B, D), jnp.float32),
        scratch_types=[
            pltpu.VMEM((BPW,), jnp.int32),
            pltpu.VMEM((CHUNK, D), jnp.float32),
            pltpu.SemaphoreType.DMA,
        ],
    )
    def k(ids_hbm, table_hbm, out_hbm, idx_v, rows_v, gsem):
        wid = lax.axis_index("s") * NC + lax.axis_index("c")
        base = wid * BPW
        pltpu.sync_copy(ids_hbm.at[pl.ds(base, BPW)], idx_v)

        def body(c, carry):
            off = c * CHUNK
            pltpu.async_copy(
                table_hbm.at[idx_v.at[pl.ds(off, CHUNK)]], rows_v, gsem
            ).wait()
            pltpu.sync_copy(rows_v, out_hbm.at[pl.ds(base + off, CHUNK)])
            return carry

        lax.fori_loop(0, NCH, body, 0)

    return k(position_ids, pos_embed)


def kernel(position_ids, pos_embed):
    return _sc_gather(position_ids.astype(jnp.int32), pos_embed)


# speedup vs baseline: 3.4586x; 1.0612x over previous
import functools

import jax
import jax.numpy as jnp
from jax import lax
from jax.experimental import pallas as pl
from jax.experimental.pallas import tpu as pltpu
from jax.experimental.pallas import tpu_sc as plsc

V = 1024
D = 384
B = 131072
NC = 2
NS = 16
NW = NC * NS
BPW = B // NW
CHUNK = 128
NCH = BPW // CHUNK


def _sc_gather(position_ids, pos_embed):
    mesh = plsc.VectorSubcoreMesh(core_axis_name="c", subcore_axis_name="s")

    @functools.partial(
        pl.kernel,
        mesh=mesh,
        out_type=jax.ShapeDtypeStruct((B, D), jnp.float32),
        scratch_types=[
            pltpu.VMEM((BPW,), jnp.int32),
            pltpu.VMEM((2, CHUNK, D), jnp.float32),
            pltpu.SemaphoreType.DMA,
            pltpu.SemaphoreType.DMA,
        ],
    )
    def k(ids_hbm, table_hbm, out_hbm, idx_v, rows_v, gsem, wsem):
        wid = lax.axis_index("s") * NC + lax.axis_index("c")
        base = wid * BPW
        pltpu.sync_copy(ids_hbm.at[pl.ds(base, BPW)], idx_v)

        def start_gather(ch, b):
            pltpu.async_copy(
                table_hbm.at[idx_v.at[pl.ds(ch * CHUNK, CHUNK)]],
                rows_v.at[b],
                gsem,
            )

        def start_write(ch, b):
            pltpu.async_copy(
                rows_v.at[b],
                out_hbm.at[pl.ds(base + ch * CHUNK, CHUNK)],
                wsem,
            )

        def wait_gather():
            pltpu.make_async_copy(
                table_hbm.at[idx_v.at[pl.ds(0, CHUNK)]], rows_v.at[0], gsem
            ).wait()

        def wait_write():
            pltpu.make_async_copy(
                rows_v.at[0], out_hbm.at[pl.ds(base, CHUNK)], wsem
            ).wait()

        start_gather(0, 0)
        wait_gather()
        start_write(0, 0)
        start_gather(1, 1)

        def pair(i, carry):
            for choff, b in ((1, 1), (2, 0)):
                ch = 2 * i + choff
                wait_gather()
                start_write(ch, b)
                wait_write()
                start_gather(ch + 1, 1 - b)
            return carry

        lax.fori_loop(0, (NCH - 2) // 2, pair, 0)

        wait_gather()
        start_write(NCH - 1, 1)
        wait_write()
        wait_write()

    return k(position_ids, pos_embed)


def kernel(position_ids, pos_embed):
    return _sc_gather(position_ids.astype(jnp.int32), pos_embed)


# speedup vs baseline: 3.5388x; 1.0232x over previous
import functools

import jax
import jax.numpy as jnp
from jax import lax
from jax.experimental import pallas as pl
from jax.experimental.pallas import tpu as pltpu
from jax.experimental.pallas import tpu_sc as plsc

V = 1024
D = 384
B = 131072
NC = 2
NS = 16
NW = NC * NS
BPW = B // NW
CHUNK = 64
NCH = BPW // CHUNK
NBUF = 4


def _sc_gather(position_ids, pos_embed):
    mesh = plsc.VectorSubcoreMesh(core_axis_name="c", subcore_axis_name="s")

    @functools.partial(
        pl.kernel,
        mesh=mesh,
        out_type=jax.ShapeDtypeStruct((B, D), jnp.float32),
        scratch_types=[
            pltpu.VMEM((BPW,), jnp.int32),
            pltpu.VMEM((NBUF, CHUNK, D), jnp.float32),
            pltpu.SemaphoreType.DMA,
            pltpu.SemaphoreType.DMA,
        ],
    )
    def k(ids_hbm, table_hbm, out_hbm, idx_v, rows_v, gsem, wsem):
        wid = lax.axis_index("s") * NC + lax.axis_index("c")
        base = wid * BPW
        pltpu.sync_copy(ids_hbm.at[pl.ds(base, BPW)], idx_v)

        def start_gather(ch, b):
            pltpu.async_copy(
                table_hbm.at[idx_v.at[pl.ds(ch * CHUNK, CHUNK)]],
                rows_v.at[b],
                gsem,
            )

        def start_write(ch, b):
            pltpu.async_copy(
                rows_v.at[b],
                out_hbm.at[pl.ds(base + ch * CHUNK, CHUNK)],
                wsem,
            )

        def wait_gather():
            pltpu.make_async_copy(
                table_hbm.at[idx_v.at[pl.ds(0, CHUNK)]], rows_v.at[0], gsem
            ).wait()

        def wait_write():
            pltpu.make_async_copy(
                rows_v.at[0], out_hbm.at[pl.ds(base, CHUNK)], wsem
            ).wait()

        start_gather(0, 0)
        start_gather(1, 1)
        for ch in (0, 1):
            wait_gather()
            start_write(ch, ch % NBUF)
            start_gather(ch + 2, (ch + 2) % NBUF)

        def quad(i, carry):
            ch0 = 4 * i + 2
            for p in range(4):
                ch = ch0 + p
                wait_gather()
                start_write(ch, (2 + p) % NBUF)
                wait_write()
                start_gather(ch + 2, p)
            return carry

        lax.fori_loop(0, (NCH - 4) // 4, quad, 0)

        for ch in (NCH - 2, NCH - 1):
            wait_gather()
            start_write(ch, ch % NBUF)
            wait_write()
        wait_write()
        wait_write()

    return k(position_ids, pos_embed)


def kernel(position_ids, pos_embed):
    return _sc_gather(position_ids.astype(jnp.int32), pos_embed)
